# baseline (speedup 1.0000x reference)
import jax
import jax.numpy as jnp
from jax.experimental import pallas as pl


TILE_N = 512


def _pq_encode_kernel(x_ref, at_ref, b_ref, ct_ref, n2_ref, out_ref):
    t = x_ref.shape[0]
    m_total, dsub, ksub = ct_ref.shape
    y = jnp.dot(x_ref[...], at_ref[...], preferred_element_type=jnp.float32)
    y = y + b_ref[...]
    lane = jax.lax.broadcasted_iota(jnp.int32, (t, ksub), 1)
    cols = []
    for m in range(m_total):
        ym = y[:, m * dsub:(m + 1) * dsub]
        dot = jnp.dot(ym, ct_ref[m], preferred_element_type=jnp.float32)
        dis = n2_ref[m][None, :] - 2.0 * dot
        mn = jnp.min(dis, axis=1, keepdims=True)
        idx = jnp.where(dis == mn, lane, ksub)
        cols.append(jnp.min(idx, axis=1, keepdims=True))
    out_ref[...] = jnp.concatenate(cols, axis=1)


def kernel(x, A, b, centroids, norm2):
    n, d = x.shape
    m_total, ksub, dsub = centroids.shape
    at = A.T
    ct = jnp.transpose(centroids, (0, 2, 1))
    b2 = b.reshape(1, d)
    grid = (n // TILE_N,)
    out = pl.pallas_call(
        _pq_encode_kernel,
        grid=grid,
        in_specs=[
            pl.BlockSpec((TILE_N, d), lambda i: (i, 0)),
            pl.BlockSpec((d, d), lambda i: (0, 0)),
            pl.BlockSpec((1, d), lambda i: (0, 0)),
            pl.BlockSpec((m_total, dsub, ksub), lambda i: (0, 0, 0)),
            pl.BlockSpec((m_total, ksub), lambda i: (0, 0)),
        ],
        out_specs=pl.BlockSpec((TILE_N, m_total), lambda i: (i, 0)),
        out_shape=jax.ShapeDtypeStruct((n, m_total), jnp.int32),
    )(x, at, b2, ct, norm2)
    return out.astype(jnp.uint8)


# speedup vs baseline: 3.0430x; 3.0430x over previous
import jax
import jax.numpy as jnp
from jax.experimental import pallas as pl


TILE_N = 512


def _pq_encode_kernel(x_ref, at_ref, b_ref, cbd_ref, n2_ref, out_ref):
    t = x_ref.shape[0]
    groups, khalf, nwide = cbd_ref.shape
    ksub = 256
    m_per_g = nwide // ksub
    y = jnp.dot(x_ref[...], at_ref[...], preferred_element_type=jnp.float32)
    y = y + b_ref[...]
    lane_f = jax.lax.broadcasted_iota(jnp.int32, (t, ksub), 1).astype(jnp.float32)
    cols = []
    for g in range(groups):
        dis = jnp.dot(y[:, g * khalf:(g + 1) * khalf], cbd_ref[g],
                      preferred_element_type=jnp.float32)
        dis = dis + n2_ref[0, g * nwide:(g + 1) * nwide][None, :]
        for mm in range(m_per_g):
            d = dis[:, mm * ksub:(mm + 1) * ksub]
            mn = jnp.min(d, axis=1, keepdims=True)
            idxf = jnp.where(d == mn, lane_f, float(ksub))
            cols.append(jnp.min(idxf, axis=1, keepdims=True))
    out_ref[...] = jnp.concatenate(cols, axis=1).astype(jnp.int32)


def kernel(x, A, b, centroids, norm2):
    n, d = x.shape
    m_total, ksub, dsub = centroids.shape
    at = A.T
    b2 = b.reshape(1, d)
    cbd4 = jnp.einsum("mz,mkd->mdzk", jnp.eye(m_total, dtype=jnp.float32),
                      -2.0 * centroids)
    half = m_total // 2
    c_full = cbd4.reshape(m_total * dsub, m_total * ksub)
    cbd = jnp.stack([c_full[:half * dsub, :half * ksub],
                     c_full[half * dsub:, half * ksub:]])
    n2 = norm2.reshape(1, m_total * ksub)
    grid = (n // TILE_N,)
    out = pl.pallas_call(
        _pq_encode_kernel,
        grid=grid,
        in_specs=[
            pl.BlockSpec((TILE_N, d), lambda i: (i, 0)),
            pl.BlockSpec((d, d), lambda i: (0, 0)),
            pl.BlockSpec((1, d), lambda i: (0, 0)),
            pl.BlockSpec((2, half * dsub, half * ksub), lambda i: (0, 0, 0)),
            pl.BlockSpec((1, m_total * ksub), lambda i: (0, 0)),
        ],
        out_specs=pl.BlockSpec((TILE_N, m_total), lambda i: (i, 0)),
        out_shape=jax.ShapeDtypeStruct((n, m_total), jnp.int32),
    )(x, at, b2, cbd, n2)
    return out.astype(jnp.uint8)


# speedup vs baseline: 3.2078x; 1.0541x over previous
import jax
import jax.numpy as jnp
from jax.experimental import pallas as pl


TILE_N = 512


def _pq_encode_kernel(x_ref, at_ref, b_ref, cbd_ref, n2_ref, e_ref, out_ref):
    t = x_ref.shape[0]
    groups, khalf, nwide = cbd_ref.shape
    ksub = 256
    m_per_g = nwide // ksub
    y = jnp.dot(x_ref[...], at_ref[...], preferred_element_type=jnp.float32)
    y = y + b_ref[...]
    codes = []
    for g in range(groups):
        dis = jnp.dot(y[:, g * khalf:(g + 1) * khalf], cbd_ref[g],
                      preferred_element_type=jnp.float32)
        dis = dis + n2_ref[0, g * nwide:(g + 1) * nwide][None, :]
        masks = []
        for mm in range(m_per_g):
            d = dis[:, mm * ksub:(mm + 1) * ksub]
            mn = jnp.min(d, axis=1, keepdims=True)
            masks.append(jnp.where(d == mn, 1.0, 0.0).astype(jnp.bfloat16))
        mask_g = jnp.concatenate(masks, axis=1)
        codes.append(jnp.dot(mask_g, e_ref[g],
                             preferred_element_type=jnp.float32))
    out_ref[...] = jnp.concatenate(codes, axis=1).astype(jnp.int32)


def kernel(x, A, b, centroids, norm2):
    n, d = x.shape
    m_total, ksub, dsub = centroids.shape
    at = A.T
    b2 = b.reshape(1, d)
    cbd4 = jnp.einsum("mz,mkd->mdzk", jnp.eye(m_total, dtype=jnp.float32),
                      -2.0 * centroids)
    half = m_total // 2
    c_full = cbd4.reshape(m_total * dsub, m_total * ksub)
    cbd = jnp.stack([c_full[:half * dsub, :half * ksub],
                     c_full[half * dsub:, half * ksub:]])
    n2 = norm2.reshape(1, m_total * ksub)
    lane = jnp.arange(ksub, dtype=jnp.float32)
    e4 = jnp.einsum("mz,k->mkz", jnp.eye(half, dtype=jnp.float32), lane)
    e = jnp.stack([e4.reshape(half * ksub, half)] * 2).astype(jnp.bfloat16)
    grid = (n // TILE_N,)
    out = pl.pallas_call(
        _pq_encode_kernel,
        grid=grid,
        in_specs=[
            pl.BlockSpec((TILE_N, d), lambda i: (i, 0)),
            pl.BlockSpec((d, d), lambda i: (0, 0)),
            pl.BlockSpec((1, d), lambda i: (0, 0)),
            pl.BlockSpec((2, half * dsub, half * ksub), lambda i: (0, 0, 0)),
            pl.BlockSpec((1, m_total * ksub), lambda i: (0, 0)),
            pl.BlockSpec((2, half * ksub, half), lambda i: (0, 0, 0)),
        ],
        out_specs=pl.BlockSpec((TILE_N, m_total), lambda i: (i, 0)),
        out_shape=jax.ShapeDtypeStruct((n, m_total), jnp.int32),
    )(x, at, b2, cbd, n2, e)
    return out.astype(jnp.uint8)


# speedup vs baseline: 3.8169x; 1.1899x over previous
import jax
import jax.numpy as jnp
from jax.experimental import pallas as pl


TILE_N = 512
EX_J_PER_G = (13, 0)


def _pq_encode_kernel(x_ref, at_ref, b_ref, cbd_ref, n2_ref, e_ref, out_ref):
    t = x_ref.shape[0]
    groups, khalf, nwide = cbd_ref.shape
    ksub = 256
    m_per_g = nwide // ksub
    y = jnp.dot(x_ref[...], at_ref[...], preferred_element_type=jnp.float32)
    y = y + b_ref[...]
    lane_f = jax.lax.broadcasted_iota(jnp.int32, (t, ksub), 1).astype(jnp.float32)
    codes = []
    for g in range(groups):
        dis = jnp.dot(y[:, g * khalf:(g + 1) * khalf], cbd_ref[g],
                      preferred_element_type=jnp.float32)
        dis = dis + n2_ref[0, g * nwide:(g + 1) * nwide][None, :]
        ex_j = EX_J_PER_G[g]
        if ex_j:
            masks = []
            for mm in range(ex_j):
                d = dis[:, mm * ksub:(mm + 1) * ksub]
                mn = jnp.min(d, axis=1, keepdims=True)
                masks.append(jnp.where(d == mn, 1.0, 0.0).astype(jnp.bfloat16))
            mask_g = jnp.concatenate(masks, axis=1)
            codes.append(jnp.dot(mask_g, e_ref[g][:ex_j * ksub, :ex_j],
                                 preferred_element_type=jnp.float32))
        cols = []
        for mm in range(ex_j, m_per_g):
            d = dis[:, mm * ksub:(mm + 1) * ksub]
            mn = jnp.min(d, axis=1, keepdims=True)
            idx = jnp.where(d == mn, lane_f, 0.0)
            cols.append(jnp.sum(idx, axis=1, keepdims=True))
        if cols:
            codes.append(jnp.concatenate(cols, axis=1))
    out_ref[...] = jnp.concatenate(codes, axis=1).astype(jnp.int32)


def kernel(x, A, b, centroids, norm2):
    n, d = x.shape
    m_total, ksub, dsub = centroids.shape
    at = A.T
    b2 = b.reshape(1, d)
    cbd4 = jnp.einsum("mz,mkd->mdzk", jnp.eye(m_total, dtype=jnp.float32),
                      -2.0 * centroids)
    half = m_total // 2
    c_full = cbd4.reshape(m_total * dsub, m_total * ksub)
    cbd = jnp.stack([c_full[:half * dsub, :half * ksub],
                     c_full[half * dsub:, half * ksub:]])
    n2 = norm2.reshape(1, m_total * ksub)
    lane = jnp.arange(ksub, dtype=jnp.float32)
    e4 = jnp.einsum("mz,k->mkz", jnp.eye(half, dtype=jnp.float32), lane)
    e = jnp.stack([e4.reshape(half * ksub, half)] * 2).astype(jnp.bfloat16)
    grid = (n // TILE_N,)
    out = pl.pallas_call(
        _pq_encode_kernel,
        grid=grid,
        in_specs=[
            pl.BlockSpec((TILE_N, d), lambda i: (i, 0)),
            pl.BlockSpec((d, d), lambda i: (0, 0)),
            pl.BlockSpec((1, d), lambda i: (0, 0)),
            pl.BlockSpec((2, half * dsub, half * ksub), lambda i: (0, 0, 0)),
            pl.BlockSpec((1, m_total * ksub), lambda i: (0, 0)),
            pl.BlockSpec((2, half * ksub, half), lambda i: (0, 0, 0)),
        ],
        out_specs=pl.BlockSpec((TILE_N, m_total), lambda i: (i, 0)),
        out_shape=jax.ShapeDtypeStruct((n, m_total), jnp.int32),
    )(x, at, b2, cbd, n2, e)
    return out.astype(jnp.uint8)
